# Initial kernel scaffold; baseline (speedup 1.0000x reference)
import jax
import jax.numpy as jnp
from jax.experimental import pallas as pl


def kernel(X, GP_info):
    raise NotImplementedError("write your pallas kernel here")


# speedup vs baseline: 6.5343x; 6.5343x over previous
import functools

import jax
import jax.numpy as jnp
from jax import lax
from jax.experimental import pallas as pl
from jax.experimental.pallas import tpu as pltpu
from jax.experimental.pallas import tpu_sc as plsc

S = 16384
K = 32
D = 128
LANES = 16
NC, NS = 2, 16
NW = NC * NS
SEGS_PER_W = S // NW
CHUNK = 32
HALF = CHUNK // 2
ROWS_PER_HALF = HALF * K
IDX_ROWS = CHUNK * K // 128
N_CHUNKS = SEGS_PER_W // CHUNK


def _make_kernel():
    mesh = plsc.VectorSubcoreMesh(core_axis_name="c", subcore_axis_name="s")

    @functools.partial(
        pl.kernel,
        mesh=mesh,
        out_type=jax.ShapeDtypeStruct((S, D), jnp.float32),
        scratch_types=[
            pltpu.VMEM((IDX_ROWS, 128), jnp.int32),
            pltpu.VMEM((ROWS_PER_HALF, D), jnp.float32),
            pltpu.VMEM((CHUNK, D), jnp.float32),
            pltpu.SemaphoreType.DMA,
        ],
    )
    def seg_mean(x_hbm, gp_hbm, out_hbm, idx_v, rows_v, out_v, sem):
        wid = lax.axis_index("s") * NC + lax.axis_index("c")

        def chunk_body(ci, _):
            seg_base = pl.multiple_of(wid * SEGS_PER_W + ci * CHUNK, CHUNK)
            idx_row0 = pl.multiple_of(seg_base * K // 128, IDX_ROWS)
            pltpu.sync_copy(gp_hbm.at[pl.ds(idx_row0, IDX_ROWS)], idx_v)

            for half in range(2):
                copies = []
                for g in range(IDX_ROWS // 2):
                    copies.append(
                        pltpu.async_copy(
                            x_hbm.at[idx_v.at[half * (IDX_ROWS // 2) + g]],
                            rows_v.at[pl.ds(g * 128, 128)],
                            sem,
                        )
                    )
                for cp in copies:
                    cp.wait()

                def seg_body(si, _):
                    row0 = si * K
                    accs = [rows_v[row0, pl.ds(j * LANES, LANES)]
                            for j in range(D // LANES)]
                    for r in range(1, K):
                        for j in range(D // LANES):
                            accs[j] = accs[j] + rows_v[row0 + r,
                                                       pl.ds(j * LANES,
                                                             LANES)]
                    for j in range(D // LANES):
                        out_v[half * HALF + si,
                              pl.ds(j * LANES, LANES)] = accs[j] * (1.0 / K)
                    return 0

                lax.fori_loop(0, HALF, seg_body, 0)

            pltpu.sync_copy(out_v, out_hbm.at[pl.ds(seg_base, CHUNK)])
            return 0

        lax.fori_loop(0, N_CHUNKS, chunk_body, 0)

    return seg_mean


_seg_mean = _make_kernel()


@jax.jit
def kernel(X, GP_info):
    gp = GP_info.astype(jnp.int32).reshape(S * K // 128, 128)
    return _seg_mean(X, gp)


# speedup vs baseline: 10.9426x; 1.6746x over previous
import functools

import jax
import jax.numpy as jnp
from jax import lax
from jax.experimental import pallas as pl
from jax.experimental.pallas import tpu as pltpu
from jax.experimental.pallas import tpu_sc as plsc

S = 16384
K = 32
D = 128
LANES = 16
NC, NS = 2, 16
NW = NC * NS
SEGS_PER_W = S // NW
W_IDX_ROWS = SEGS_PER_W * K // 128
STEP = 8
STEP_ROWS = STEP * K
N_STEPS = SEGS_PER_W // STEP
GATHERS_PER_STEP = STEP_ROWS // 128


def _make_kernel():
    mesh = plsc.VectorSubcoreMesh(core_axis_name="c", subcore_axis_name="s")

    @functools.partial(
        pl.kernel,
        mesh=mesh,
        out_type=jax.ShapeDtypeStruct((S, D), jnp.float32),
        scratch_types=[
            pltpu.VMEM((W_IDX_ROWS, 128), jnp.int32),
            pltpu.VMEM((2 * STEP_ROWS, D), jnp.float32),
            pltpu.VMEM((2 * 2 * STEP, D), jnp.float32),
            pltpu.SemaphoreType.DMA,
            pltpu.SemaphoreType.DMA,
            pltpu.SemaphoreType.DMA,
        ],
    )
    def seg_mean(x_hbm, gp_hbm, out_hbm, idx_v, rows_v, out_v,
                 sem_g0, sem_g1, sem_out):
        wid = lax.axis_index("s") * NC + lax.axis_index("c")
        sems = (sem_g0, sem_g1)

        pltpu.sync_copy(gp_hbm.at[pl.ds(wid * W_IDX_ROWS, W_IDX_ROWS)],
                        idx_v)

        def gather_copies(u, buf):
            base = buf * STEP_ROWS
            return [
                pltpu.make_async_copy(
                    x_hbm.at[idx_v.at[u * GATHERS_PER_STEP + h]],
                    rows_v.at[pl.ds(base + h * 128, 128)],
                    sems[buf],
                )
                for h in range(GATHERS_PER_STEP)
            ]

        def fire(u, buf):
            for cp in gather_copies(u, buf):
                cp.start()

        def drain(u, buf):
            for cp in gather_copies(u, buf):
                cp.wait()

        def compute(t, buf, out_half):
            base = buf * STEP_ROWS
            out_base = out_half * STEP

            def seg_body(s, _):
                row0 = base + s * K
                accs = [rows_v[row0, pl.ds(j * LANES, LANES)]
                        for j in range(D // LANES)]
                for r in range(1, K):
                    for j in range(D // LANES):
                        accs[j] = accs[j] + rows_v[row0 + r,
                                                   pl.ds(j * LANES, LANES)]
                for j in range(D // LANES):
                    out_v[out_base + s,
                          pl.ds(j * LANES, LANES)] = accs[j] * (1.0 / K)
                return 0

            lax.fori_loop(0, STEP, seg_body, 0)

        fire(0, 0)

        def outer_body(gi, _):
            g = gi * 2
            out_half = pl.multiple_of((gi % 2) * 2, 2)
            seg_base = pl.multiple_of(wid * SEGS_PER_W + g * STEP, 2 * STEP)
            out_store = pltpu.make_async_copy(
                out_v.at[pl.ds(out_half * STEP, 2 * STEP)],
                out_hbm.at[pl.ds(seg_base, 2 * STEP)],
                sem_out,
            )
            lax.cond(gi >= 2, out_store.wait, lambda: None)

            for b in range(2):
                t = g + b
                u = t + 1
                if b == 0:
                    fire(u, 1)
                else:
                    lax.cond(gi < N_STEPS // 2 - 1,
                             lambda: fire(u, 0), lambda: None)
                drain(t, b)
                compute(t, b, out_half + b)

            out_store.start()
            return 0

        lax.fori_loop(0, N_STEPS // 2, outer_body, 0)
        pltpu.make_async_copy(
            out_v.at[pl.ds(0, 2 * STEP)],
            out_hbm.at[pl.ds(pl.multiple_of(wid * SEGS_PER_W, 2 * STEP),
                             2 * STEP)],
            sem_out,
        ).wait()
        pltpu.make_async_copy(
            out_v.at[pl.ds(0, 2 * STEP)],
            out_hbm.at[pl.ds(pl.multiple_of(wid * SEGS_PER_W, 2 * STEP),
                             2 * STEP)],
            sem_out,
        ).wait()

    return seg_mean


_seg_mean = _make_kernel()


@jax.jit
def kernel(X, GP_info):
    gp = GP_info.astype(jnp.int32).reshape(S * K // 128, 128)
    return _seg_mean(X, gp)


# speedup vs baseline: 12.7854x; 1.1684x over previous
import functools

import jax
import jax.numpy as jnp
from jax import lax
from jax.experimental import pallas as pl
from jax.experimental.pallas import tpu as pltpu
from jax.experimental.pallas import tpu_sc as plsc

S = 16384
K = 32
D = 128
LANES = 16
NC, NS = 2, 16
NW = NC * NS
SEGS_PER_W = S // NW
W_IDX_ROWS = SEGS_PER_W * K // 128
STEP = 8
STEP_ROWS = STEP * K
N_STEPS = SEGS_PER_W // STEP
GATHERS_PER_STEP = STEP_ROWS // 128
GRP = 8


def _make_kernel():
    mesh = plsc.VectorSubcoreMesh(core_axis_name="c", subcore_axis_name="s")

    @functools.partial(
        pl.kernel,
        mesh=mesh,
        out_type=jax.ShapeDtypeStruct((S, D), jnp.float32),
        scratch_types=[
            pltpu.VMEM((W_IDX_ROWS, 128), jnp.int32),
            pltpu.VMEM((2 * STEP_ROWS, D), jnp.float32),
            pltpu.VMEM((2 * 2 * STEP, D), jnp.float32),
            pltpu.SemaphoreType.DMA,
            pltpu.SemaphoreType.DMA,
            pltpu.SemaphoreType.DMA,
        ],
    )
    def seg_mean(x_hbm, gp_hbm, out_hbm, idx_v, rows_v, out_v,
                 sem_g0, sem_g1, sem_out):
        wid = lax.axis_index("s") * NC + lax.axis_index("c")
        sems = (sem_g0, sem_g1)

        pltpu.sync_copy(gp_hbm.at[pl.ds(wid * W_IDX_ROWS, W_IDX_ROWS)],
                        idx_v)

        def gather_copies(u, buf):
            base = buf * STEP_ROWS
            return [
                pltpu.make_async_copy(
                    x_hbm.at[idx_v.at[u * GATHERS_PER_STEP + h]],
                    rows_v.at[pl.ds(base + h * 128, 128)],
                    sems[buf],
                )
                for h in range(GATHERS_PER_STEP)
            ]

        def fire(u, buf):
            for cp in gather_copies(u, buf):
                cp.start()

        def drain(u, buf):
            for cp in gather_copies(u, buf):
                cp.wait()

        def compute(t, buf, out_half):
            base = buf * STEP_ROWS
            out_base = out_half * STEP

            def seg_body(s, _):
                row0 = base + s * K

                def grp_body(g2, accs):
                    r0 = row0 + g2 * GRP
                    for r in range(GRP):
                        accs = tuple(
                            accs[j] + rows_v[r0 + r, pl.ds(j * LANES, LANES)]
                            for j in range(D // LANES)
                        )
                    return accs

                zero = jnp.zeros((LANES,), jnp.float32)
                accs = lax.fori_loop(0, K // GRP, grp_body,
                                     (zero,) * (D // LANES))
                for j in range(D // LANES):
                    out_v[out_base + s,
                          pl.ds(j * LANES, LANES)] = accs[j] * (1.0 / K)
                return 0

            lax.fori_loop(0, STEP, seg_body, 0)

        fire(0, 0)

        def outer_body(gi, _):
            g = gi * 2
            out_half = pl.multiple_of((gi % 2) * 2, 2)
            seg_base = pl.multiple_of(wid * SEGS_PER_W + g * STEP, 2 * STEP)
            out_store = pltpu.make_async_copy(
                out_v.at[pl.ds(out_half * STEP, 2 * STEP)],
                out_hbm.at[pl.ds(seg_base, 2 * STEP)],
                sem_out,
            )
            lax.cond(gi >= 2, out_store.wait, lambda: None)

            for b in range(2):
                t = g + b
                u = t + 1
                if b == 0:
                    fire(u, 1)
                else:
                    lax.cond(gi < N_STEPS // 2 - 1,
                             lambda: fire(u, 0), lambda: None)
                drain(t, b)
                compute(t, b, out_half + b)

            out_store.start()
            return 0

        lax.fori_loop(0, N_STEPS // 2, outer_body, 0)
        pltpu.make_async_copy(
            out_v.at[pl.ds(0, 2 * STEP)],
            out_hbm.at[pl.ds(pl.multiple_of(wid * SEGS_PER_W, 2 * STEP),
                             2 * STEP)],
            sem_out,
        ).wait()
        pltpu.make_async_copy(
            out_v.at[pl.ds(0, 2 * STEP)],
            out_hbm.at[pl.ds(pl.multiple_of(wid * SEGS_PER_W, 2 * STEP),
                             2 * STEP)],
            sem_out,
        ).wait()

    return seg_mean


_seg_mean = _make_kernel()


@jax.jit
def kernel(X, GP_info):
    gp = GP_info.astype(jnp.int32).reshape(S * K // 128, 128)
    return _seg_mean(X, gp)


# speedup vs baseline: 14.5137x; 1.1352x over previous
import functools

import jax
import jax.numpy as jnp
from jax import lax
from jax.experimental import pallas as pl
from jax.experimental.pallas import tpu as pltpu
from jax.experimental.pallas import tpu_sc as plsc

S = 16384
K = 32
D = 128
LANES = 16
NC, NS = 2, 16
NW = NC * NS
SEGS_PER_W = S // NW
W_IDX_ROWS = SEGS_PER_W * K // 128
STEP = 8
STEP_ROWS = STEP * K
N_STEPS = SEGS_PER_W // STEP
GATHERS_PER_STEP = STEP_ROWS // 128
GRP = 8
DEPTH = 3


def _make_kernel():
    mesh = plsc.VectorSubcoreMesh(core_axis_name="c", subcore_axis_name="s")

    @functools.partial(
        pl.kernel,
        mesh=mesh,
        out_type=jax.ShapeDtypeStruct((S, D), jnp.float32),
        scratch_types=[
            pltpu.VMEM((W_IDX_ROWS, 128), jnp.int32),
            pltpu.VMEM((DEPTH * STEP_ROWS, D), jnp.float32),
            pltpu.VMEM((DEPTH * STEP, D), jnp.float32),
            pltpu.SemaphoreType.DMA,
            pltpu.SemaphoreType.DMA,
            pltpu.SemaphoreType.DMA,
            pltpu.SemaphoreType.DMA,
        ],
    )
    def seg_mean(x_hbm, gp_hbm, out_hbm, idx_v, rows_v, out_v,
                 sem_g0, sem_g1, sem_g2, sem_out):
        wid = lax.axis_index("s") * NC + lax.axis_index("c")
        sems = (sem_g0, sem_g1, sem_g2)

        pltpu.sync_copy(gp_hbm.at[pl.ds(wid * W_IDX_ROWS, W_IDX_ROWS)],
                        idx_v)

        def gather_copies(u, buf):
            base = buf * STEP_ROWS
            return [
                pltpu.make_async_copy(
                    x_hbm.at[idx_v.at[u * GATHERS_PER_STEP + h]],
                    rows_v.at[pl.ds(base + h * 128, 128)],
                    sems[buf],
                )
                for h in range(GATHERS_PER_STEP)
            ]

        def fire(u, buf):
            for cp in gather_copies(u, buf):
                cp.start()

        def drain(u, buf):
            for cp in gather_copies(u, buf):
                cp.wait()

        def out_store(t, buf):
            seg_base = pl.multiple_of(wid * SEGS_PER_W + t * STEP, STEP)
            return pltpu.make_async_copy(
                out_v.at[pl.ds(buf * STEP, STEP)],
                out_hbm.at[pl.ds(seg_base, STEP)],
                sem_out,
            )

        def compute(t, buf):
            base = buf * STEP_ROWS
            out_base = buf * STEP

            def seg_body(s, _):
                row0 = base + s * K

                def grp_body(g2, accs):
                    r0 = row0 + g2 * GRP
                    for r in range(GRP):
                        accs = tuple(
                            accs[j] + rows_v[r0 + r, pl.ds(j * LANES, LANES)]
                            for j in range(D // LANES)
                        )
                    return accs

                zero = jnp.zeros((LANES,), jnp.float32)
                accs = lax.fori_loop(0, K // GRP, grp_body,
                                     (zero,) * (D // LANES))
                for j in range(D // LANES):
                    out_v[out_base + s,
                          pl.ds(j * LANES, LANES)] = accs[j] * (1.0 / K)
                return 0

            lax.fori_loop(0, STEP, seg_body, 0)

        def full_step(t, buf, fire_pred, wait_pred):
            lax.cond(fire_pred,
                     lambda: fire(t + 2, (buf + 2) % DEPTH), lambda: None)
            drain(t, buf)
            lax.cond(wait_pred, out_store(t, buf).wait, lambda: None)
            compute(t, buf)
            out_store(t, buf).start()

        fire(0, 0)
        fire(1, 1)

        def outer_body(gi, _):
            for b in range(DEPTH):
                t = gi * DEPTH + b
                fire_pred = (gi < N_STEPS // DEPTH - 1) if b == 2 else True
                full_step(t, b, fire_pred, gi >= 1)
            return 0

        lax.fori_loop(0, N_STEPS // DEPTH, outer_body, 0)
        full_step(N_STEPS - 1, 0, False, True)

        for t in (N_STEPS - 3, N_STEPS - 2, N_STEPS - 1):
            out_store(t, t % DEPTH).wait()

    return seg_mean


_seg_mean = _make_kernel()


@jax.jit
def kernel(X, GP_info):
    gp = GP_info.astype(jnp.int32).reshape(S * K // 128, 128)
    return _seg_mean(X, gp)
